# baseline (speedup 1.0000x reference)
import functools

import jax
import jax.numpy as jnp
from jax import lax
from jax.experimental import pallas as pl
from jax.experimental.pallas import tpu as pltpu
from jax.experimental.pallas import tpu_sc as plsc

NUM_USERS = 25000
NUM_ITEMS = 25000
N = NUM_USERS + NUM_ITEMS
D = 64
NNZ = 800000
B = 4096

NC = 2
NS = 16
R_T = 1568
R_C = NS * R_T
N_P = NC * R_C
ACC_ROWS = R_C + 8

G = 128
GPS = 28
SUP = GPS * G
NSUP = 14
EPS = NSUP * SUP
NNZ_P = NS * EPS
CROWS_PER_SUB = EPS // G


def _make_spmm():
  mesh = plsc.VectorSubcoreMesh(core_axis_name="c", subcore_axis_name="s")

  @functools.partial(
      pl.kernel,
      out_type=jax.ShapeDtypeStruct((N_P, D), jnp.float32),
      mesh=mesh,
      compiler_params=pltpu.CompilerParams(use_tc_tiling_on_sc=False),
      scratch_types=[
          pltpu.VMEM((SUP,), jnp.int32),
          pltpu.VMEM((SUP,), jnp.int32),
          pltpu.VMEM((SUP,), jnp.float32),
          pltpu.VMEM((2, G), jnp.int32),
          pltpu.VMEM((G, D), jnp.float32),
          pltpu.VMEM((G, D), jnp.float32),
          pltpu.VMEM_SHARED((ACC_ROWS, D), jnp.float32),
          pltpu.SemaphoreType.DMA,
          pltpu.SemaphoreType.DMA,
          pltpu.SemaphoreType.DMA,
          pltpu.SemaphoreType.DMA,
      ],
  )
  def spmm(feats_hbm, rows_hbm, cols_hbm, vals_hbm, z_hbm, out_hbm,
           rows_v, cols_v, vals_v, lidx_v, gbuf_a, gbuf_b, acc,
           gsem_a, gsem_b, ssem_a, ssem_b):
    c = lax.axis_index("c")
    s = lax.axis_index("s")
    cbase = c * R_C

    pltpu.sync_copy(z_hbm, acc.at[pl.ds(s * R_T, R_T)])
    plsc.subcore_barrier()

    bufs = ((gbuf_a, gsem_a, ssem_a), (gbuf_b, gsem_b, ssem_b))

    def issue(g, buf, sem):
      pltpu.async_copy(feats_hbm.at[cols_v.at[pl.ds(g * G, G)]], buf, sem)

    def drain(buf, sem):
      pltpu.make_async_copy(feats_hbm.at[pl.ds(0, G)], buf, sem).wait()

    def process(g, buf, bsel):
      for i in range(G // 16):
        r = rows_v[pl.ds(g * G + i * 16, 16)]
        l = r - cbase
        ok = (l >= 0) & (l < R_C)
        lidx_v[bsel, pl.ds(i * 16, 16)] = jnp.where(ok, l, R_C)
      @plsc.parallel_loop(0, G // 16, unroll=2)
      def _(i):
        v16 = vals_v[pl.ds(g * G + i * 16, 16)]
        for k in range(16):
          e = i * 16 + k
          v = v16[k]
          for j in range(D // 16):
            buf[e, pl.ds(j * 16, 16)] = buf[e, pl.ds(j * 16, 16)] * v
      pltpu.async_copy(buf, acc.at[lidx_v.at[bsel]], bufs[bsel][2], add=True)

    ebase = s * EPS
    for sup in range(NSUP):
      eoff = ebase + sup * SUP
      pltpu.sync_copy(rows_hbm.at[pl.ds(eoff, SUP)], rows_v)
      pltpu.sync_copy(cols_hbm.at[pl.ds(eoff, SUP)], cols_v)
      pltpu.sync_copy(vals_hbm.at[pl.ds(eoff, SUP)], vals_v)

      issue(0, gbuf_a, gsem_a)

      def pair(i, _):
        for b in range(2):
          g = i * 2 + b
          buf, sem, _ssem = bufs[b]
          obuf, osem, ossem = bufs[1 - b]
          drain(buf, sem)

          @pl.when(g >= 1)
          def _():
            drain(obuf, ossem)

          @pl.when(g + 1 < GPS)
          def _():
            issue(g + 1, obuf, osem)

          process(g, buf, b)
        return 0

      lax.fori_loop(0, GPS // 2, pair, 0)
      drain(gbuf_b, ssem_b)

    plsc.subcore_barrier()
    pltpu.sync_copy(acc.at[pl.ds(s * R_T, R_T)],
                    out_hbm.at[pl.ds(cbase + s * R_T, R_T)])

  return spmm


def _dense_layer(le, f, w1, b1, w2, b2):
  blk = 512

  def body(le_ref, f_ref, w1_ref, b1_ref, w2_ref, b2_ref, o_ref):
    le_v = le_ref[...]
    f_v = f_ref[...]
    acc = jnp.dot(le_v + f_v, w1_ref[...], preferred_element_type=jnp.float32)
    acc += jnp.dot(le_v * f_v, w2_ref[...], preferred_element_type=jnp.float32)
    acc += b1_ref[...] + b2_ref[...]
    o_ref[...] = jnp.where(acc > 0, acc, 0.01 * acc)

  return pl.pallas_call(
      body,
      grid=(N_P // blk,),
      in_specs=[
          pl.BlockSpec((blk, D), lambda i: (i, 0)),
          pl.BlockSpec((blk, D), lambda i: (i, 0)),
          pl.BlockSpec((D, D), lambda i: (0, 0)),
          pl.BlockSpec((1, D), lambda i: (0, 0)),
          pl.BlockSpec((D, D), lambda i: (0, 0)),
          pl.BlockSpec((1, D), lambda i: (0, 0)),
      ],
      out_specs=pl.BlockSpec((blk, D), lambda i: (i, 0)),
      out_shape=jax.ShapeDtypeStruct((N_P, D), jnp.float32),
  )(le, f, w1, b1.reshape(1, D), w2, b2.reshape(1, D))


def _make_gather():
  mesh = plsc.VectorSubcoreMesh(core_axis_name="c", subcore_axis_name="s")
  ch = B // (NC * NS)

  @functools.partial(
      pl.kernel,
      out_type=[jax.ShapeDtypeStruct((B, D), jnp.float32)] * 6,
      mesh=mesh,
      compiler_params=pltpu.CompilerParams(use_tc_tiling_on_sc=False),
      scratch_types=[
          pltpu.VMEM((ch,), jnp.int32),
          pltpu.VMEM((ch, D), jnp.float32),
          pltpu.SemaphoreType.DMA,
      ],
  )
  def gather(f0, f1, f2, u_hbm, i_hbm, o0, o1, o2, o3, o4, o5,
             idx_v, buf, sem):
    c = lax.axis_index("c")
    s = lax.axis_index("s")
    base = (s * NC + c) * ch

    pltpu.sync_copy(u_hbm.at[pl.ds(base, ch)], idx_v)
    for f, o in ((f0, o0), (f1, o1), (f2, o2)):
      pltpu.async_copy(f.at[idx_v], buf, sem).wait()
      pltpu.sync_copy(buf, o.at[pl.ds(base, ch)])

    pltpu.sync_copy(i_hbm.at[pl.ds(base, ch)], idx_v)
    for i in range(ch // 16):
      idx_v[pl.ds(i * 16, 16)] = idx_v[pl.ds(i * 16, 16)] + NUM_USERS
    for f, o in ((f0, o3), (f1, o4), (f2, o5)):
      pltpu.async_copy(f.at[idx_v], buf, sem).wait()
      pltpu.sync_copy(buf, o.at[pl.ds(base, ch)])

  return gather


def _mlp(parts, fc1_w, fc1_b, fc2_w, fc2_b, fc3_w, fc3_b):
  blk = 512
  fc3_wp = jnp.concatenate([fc3_w, jnp.zeros((32, 7), jnp.float32)], axis=1)
  fc3_bp = jnp.concatenate([fc3_b, jnp.zeros((7,), jnp.float32)])

  def body(p0, p1, p2, p3, p4, p5, w1_ref, b1_ref, w2_ref, b2_ref,
           w3_ref, b3_ref, o_ref):
    w1 = w1_ref[...]
    acc = jnp.zeros((blk, 64), jnp.float32)
    for i, p in enumerate((p0, p1, p2, p3, p4, p5)):
      acc += jnp.dot(p[...], w1[i * 64:(i + 1) * 64, :],
                     preferred_element_type=jnp.float32)
    h1 = jnp.maximum(acc + b1_ref[...], 0.0)
    h2 = jnp.maximum(
        jnp.dot(h1, w2_ref[...], preferred_element_type=jnp.float32)
        + b2_ref[...], 0.0)
    o_ref[...] = (jnp.dot(h2, w3_ref[...], preferred_element_type=jnp.float32)
                  + b3_ref[...])

  part_spec = pl.BlockSpec((blk, D), lambda i: (i, 0))
  return pl.pallas_call(
      body,
      grid=(B // blk,),
      in_specs=[part_spec] * 6 + [
          pl.BlockSpec((6 * D, D), lambda i: (0, 0)),
          pl.BlockSpec((1, D), lambda i: (0, 0)),
          pl.BlockSpec((D, 32), lambda i: (0, 0)),
          pl.BlockSpec((1, 32), lambda i: (0, 0)),
          pl.BlockSpec((32, 8), lambda i: (0, 0)),
          pl.BlockSpec((1, 8), lambda i: (0, 0)),
      ],
      out_specs=pl.BlockSpec((blk, 8), lambda i: (i, 0)),
      out_shape=jax.ShapeDtypeStruct((B, 8), jnp.float32),
  )(*parts, fc1_w, fc1_b.reshape(1, D), fc2_w, fc2_b.reshape(1, 32),
    fc3_wp, fc3_bp.reshape(1, 8))


def kernel(uids, iids, user_emb, item_emb, L_rows, L_cols, L_vals,
           W1_0, b1_0, W2_0, b2_0, W1_1, b1_1, W2_1, b2_1,
           fc1_w, fc1_b, fc2_w, fc2_b, fc3_w, fc3_b):
  feats0 = jnp.concatenate(
      [user_emb, item_emb, jnp.zeros((N_P - N, D), jnp.float32)], axis=0)
  pad_e = NNZ_P - NNZ
  rows_p = jnp.concatenate(
      [L_rows.astype(jnp.int32), jnp.zeros((pad_e,), jnp.int32)])
  cols_p = jnp.concatenate(
      [L_cols.astype(jnp.int32), jnp.zeros((pad_e,), jnp.int32)])
  vals_p = jnp.concatenate(
      [L_vals, jnp.zeros((pad_e,), jnp.float32)])
  ztile = jnp.zeros((R_T, D), jnp.float32)

  spmm = _make_spmm()
  le0 = spmm(feats0, rows_p, cols_p, vals_p, ztile)
  f1 = _dense_layer(le0, feats0, W1_0, b1_0, W2_0, b2_0)
  le1 = spmm(f1, rows_p, cols_p, vals_p, ztile)
  f2 = _dense_layer(le1, f1, W1_1, b1_1, W2_1, b2_1)

  parts = _make_gather()(feats0, f1, f2,
                         uids.astype(jnp.int32), iids.astype(jnp.int32))
  out8 = _mlp(parts, fc1_w, fc1_b, fc2_w, fc2_b, fc3_w, fc3_b)
  return out8[:, 0]


# speedup vs baseline: 1.2596x; 1.2596x over previous
import functools

import jax
import jax.numpy as jnp
from jax import lax
from jax.experimental import pallas as pl
from jax.experimental.pallas import tpu as pltpu
from jax.experimental.pallas import tpu_sc as plsc

NUM_USERS = 25000
NUM_ITEMS = 25000
N = NUM_USERS + NUM_ITEMS
D = 64
DH = D // 2
NNZ = 800000
B = 4096

NC = 2
NS = 16
N_P = 50176
R_T = N_P // NS

G = 128
GPS = 28
SUP = GPS * G
NSUP = 14
EPS = NSUP * SUP
NNZ_P = NS * EPS


def _make_spmm():
  mesh = plsc.VectorSubcoreMesh(core_axis_name="c", subcore_axis_name="s")

  @functools.partial(
      pl.kernel,
      out_type=jax.ShapeDtypeStruct((NC * N_P, DH), jnp.float32),
      mesh=mesh,
      compiler_params=pltpu.CompilerParams(use_tc_tiling_on_sc=False),
      scratch_types=[
          pltpu.VMEM((SUP,), jnp.int32),
          pltpu.VMEM((SUP,), jnp.int32),
          pltpu.VMEM((SUP,), jnp.float32),
          pltpu.VMEM((2, G), jnp.int32),
          pltpu.VMEM((G, DH), jnp.float32),
          pltpu.VMEM((G, DH), jnp.float32),
          pltpu.VMEM_SHARED((N_P, DH), jnp.float32),
          pltpu.SemaphoreType.DMA,
          pltpu.SemaphoreType.DMA,
          pltpu.SemaphoreType.DMA,
          pltpu.SemaphoreType.DMA,
      ],
  )
  def spmm(feats_hbm, rows_hbm, cols_hbm, vals_hbm, z_hbm, out_hbm,
           rows_v, cols_v, vals_v, lidx_v, gbuf_a, gbuf_b, acc,
           gsem_a, gsem_b, ssem_a, ssem_b):
    c = lax.axis_index("c")
    s = lax.axis_index("s")
    tbase = c * N_P

    pltpu.sync_copy(z_hbm, acc.at[pl.ds(s * R_T, R_T)])
    plsc.subcore_barrier()

    bufs = ((gbuf_a, gsem_a, ssem_a), (gbuf_b, gsem_b, ssem_b))

    def issue(g, buf, sem):
      pltpu.async_copy(feats_hbm.at[cols_v.at[pl.ds(g * G, G)]], buf, sem)

    def drain(buf, sem):
      pltpu.make_async_copy(feats_hbm.at[pl.ds(0, G)], buf, sem).wait()

    def process(g, buf, bsel):
      @plsc.parallel_loop(0, G // 16)
      def _(i):
        lidx_v[bsel, pl.ds(i * 16, 16)] = rows_v[pl.ds(g * G + i * 16, 16)]
      @plsc.parallel_loop(0, G // 16, unroll=2)
      def _(i):
        v16 = vals_v[pl.ds(g * G + i * 16, 16)]
        for k in range(16):
          e = i * 16 + k
          v = v16[k]
          for j in range(DH // 16):
            buf[e, pl.ds(j * 16, 16)] = buf[e, pl.ds(j * 16, 16)] * v
      pltpu.async_copy(buf, acc.at[lidx_v.at[bsel]], bufs[bsel][2], add=True)

    ebase = s * EPS
    for sup in range(NSUP):
      eoff = ebase + sup * SUP
      pltpu.sync_copy(rows_hbm.at[pl.ds(eoff, SUP)], rows_v)
      pltpu.sync_copy(cols_hbm.at[pl.ds(eoff, SUP)], cols_v)
      pltpu.sync_copy(vals_hbm.at[pl.ds(eoff, SUP)], vals_v)

      @plsc.parallel_loop(0, SUP // 16, unroll=4)
      def _(i):
        cols_v[pl.ds(i * 16, 16)] = cols_v[pl.ds(i * 16, 16)] + tbase

      issue(0, gbuf_a, gsem_a)

      def pair(i, _):
        for b in range(2):
          g = i * 2 + b
          buf, sem, _ssem = bufs[b]
          obuf, osem, ossem = bufs[1 - b]
          drain(buf, sem)

          @pl.when(g >= 1)
          def _():
            drain(obuf, ossem)

          @pl.when(g + 1 < GPS)
          def _():
            issue(g + 1, obuf, osem)

          process(g, buf, b)
        return 0

      lax.fori_loop(0, GPS // 2, pair, 0)
      drain(gbuf_b, ssem_b)

    plsc.subcore_barrier()
    pltpu.sync_copy(acc.at[pl.ds(s * R_T, R_T)],
                    out_hbm.at[pl.ds(c * N_P + s * R_T, R_T)])

  return spmm


def _dense_layer(le_flat, f, w1, b1, w2, b2):
  blk = 512

  def body(lo_ref, hi_ref, f_ref, w1_ref, b1_ref, w2_ref, b2_ref,
           o_ref, os_ref):
    le_v = jnp.concatenate([lo_ref[...], hi_ref[...]], axis=-1)
    f_v = f_ref[...]
    acc = jnp.dot(le_v + f_v, w1_ref[...], preferred_element_type=jnp.float32)
    acc += jnp.dot(le_v * f_v, w2_ref[...], preferred_element_type=jnp.float32)
    acc += b1_ref[...] + b2_ref[...]
    x = jnp.where(acc > 0, acc, 0.01 * acc)
    o_ref[...] = x
    os_ref[0] = x[:, :DH]
    os_ref[1] = x[:, DH:]

  return pl.pallas_call(
      body,
      grid=(N_P // blk,),
      in_specs=[
          pl.BlockSpec((blk, DH), lambda i: (i, 0)),
          pl.BlockSpec((blk, DH), lambda i: (N_P // 512 + i, 0)),
          pl.BlockSpec((blk, D), lambda i: (i, 0)),
          pl.BlockSpec((D, D), lambda i: (0, 0)),
          pl.BlockSpec((1, D), lambda i: (0, 0)),
          pl.BlockSpec((D, D), lambda i: (0, 0)),
          pl.BlockSpec((1, D), lambda i: (0, 0)),
      ],
      out_specs=[
          pl.BlockSpec((blk, D), lambda i: (i, 0)),
          pl.BlockSpec((2, blk, DH), lambda i: (0, i, 0)),
      ],
      out_shape=[
          jax.ShapeDtypeStruct((N_P, D), jnp.float32),
          jax.ShapeDtypeStruct((2, N_P, DH), jnp.float32),
      ],
  )(le_flat, le_flat, f, w1, b1.reshape(1, D), w2, b2.reshape(1, D))


def _make_gather():
  mesh = plsc.VectorSubcoreMesh(core_axis_name="c", subcore_axis_name="s")
  ch = B // (NC * NS)

  @functools.partial(
      pl.kernel,
      out_type=[jax.ShapeDtypeStruct((B, D), jnp.float32)] * 6,
      mesh=mesh,
      compiler_params=pltpu.CompilerParams(use_tc_tiling_on_sc=False),
      scratch_types=[
          pltpu.VMEM((ch,), jnp.int32),
          pltpu.VMEM((ch, D), jnp.float32),
          pltpu.SemaphoreType.DMA,
      ],
  )
  def gather(f0, f1, f2, u_hbm, i_hbm, o0, o1, o2, o3, o4, o5,
             idx_v, buf, sem):
    c = lax.axis_index("c")
    s = lax.axis_index("s")
    base = (s * NC + c) * ch

    pltpu.sync_copy(u_hbm.at[pl.ds(base, ch)], idx_v)
    for f, o in ((f0, o0), (f1, o1), (f2, o2)):
      pltpu.async_copy(f.at[idx_v], buf, sem).wait()
      pltpu.sync_copy(buf, o.at[pl.ds(base, ch)])

    pltpu.sync_copy(i_hbm.at[pl.ds(base, ch)], idx_v)
    for i in range(ch // 16):
      idx_v[pl.ds(i * 16, 16)] = idx_v[pl.ds(i * 16, 16)] + NUM_USERS
    for f, o in ((f0, o3), (f1, o4), (f2, o5)):
      pltpu.async_copy(f.at[idx_v], buf, sem).wait()
      pltpu.sync_copy(buf, o.at[pl.ds(base, ch)])

  return gather


def _mlp(parts, fc1_w, fc1_b, fc2_w, fc2_b, fc3_w, fc3_b):
  blk = 512
  fc3_wp = jnp.concatenate([fc3_w, jnp.zeros((32, 7), jnp.float32)], axis=1)
  fc3_bp = jnp.concatenate([fc3_b, jnp.zeros((7,), jnp.float32)])

  def body(p0, p1, p2, p3, p4, p5, w1_ref, b1_ref, w2_ref, b2_ref,
           w3_ref, b3_ref, o_ref):
    w1 = w1_ref[...]
    acc = jnp.zeros((blk, 64), jnp.float32)
    for i, p in enumerate((p0, p1, p2, p3, p4, p5)):
      acc += jnp.dot(p[...], w1[i * 64:(i + 1) * 64, :],
                     preferred_element_type=jnp.float32)
    h1 = jnp.maximum(acc + b1_ref[...], 0.0)
    h2 = jnp.maximum(
        jnp.dot(h1, w2_ref[...], preferred_element_type=jnp.float32)
        + b2_ref[...], 0.0)
    o_ref[...] = (jnp.dot(h2, w3_ref[...], preferred_element_type=jnp.float32)
                  + b3_ref[...])

  part_spec = pl.BlockSpec((blk, D), lambda i: (i, 0))
  return pl.pallas_call(
      body,
      grid=(B // blk,),
      in_specs=[part_spec] * 6 + [
          pl.BlockSpec((6 * D, D), lambda i: (0, 0)),
          pl.BlockSpec((1, D), lambda i: (0, 0)),
          pl.BlockSpec((D, 32), lambda i: (0, 0)),
          pl.BlockSpec((1, 32), lambda i: (0, 0)),
          pl.BlockSpec((32, 8), lambda i: (0, 0)),
          pl.BlockSpec((1, 8), lambda i: (0, 0)),
      ],
      out_specs=pl.BlockSpec((blk, 8), lambda i: (i, 0)),
      out_shape=jax.ShapeDtypeStruct((B, 8), jnp.float32),
  )(*parts, fc1_w, fc1_b.reshape(1, D), fc2_w, fc2_b.reshape(1, 32),
    fc3_wp, fc3_bp.reshape(1, 8))


def kernel(uids, iids, user_emb, item_emb, L_rows, L_cols, L_vals,
           W1_0, b1_0, W2_0, b2_0, W1_1, b1_1, W2_1, b2_1,
           fc1_w, fc1_b, fc2_w, fc2_b, fc3_w, fc3_b):
  feats0 = jnp.concatenate(
      [user_emb, item_emb, jnp.zeros((N_P - N, D), jnp.float32)], axis=0)
  feats0_flat = jnp.concatenate([feats0[:, :DH], feats0[:, DH:]], axis=0)
  pad_e = NNZ_P - NNZ
  rows_p = jnp.concatenate(
      [L_rows.astype(jnp.int32), jnp.zeros((pad_e,), jnp.int32)])
  cols_p = jnp.concatenate(
      [L_cols.astype(jnp.int32), jnp.zeros((pad_e,), jnp.int32)])
  vals_p = jnp.concatenate(
      [L_vals, jnp.zeros((pad_e,), jnp.float32)])
  ztile = jnp.zeros((R_T, DH), jnp.float32)

  spmm = _make_spmm()
  le0 = spmm(feats0_flat, rows_p, cols_p, vals_p, ztile)
  f1, f1_split = _dense_layer(le0, feats0, W1_0, b1_0, W2_0, b2_0)
  le1 = spmm(f1_split.reshape(NC * N_P, DH), rows_p, cols_p, vals_p, ztile)
  f2, _ = _dense_layer(le1, f1, W1_1, b1_1, W2_1, b2_1)

  parts = _make_gather()(feats0, f1, f2,
                         uids.astype(jnp.int32), iids.astype(jnp.int32))
  out8 = _mlp(parts, fc1_w, fc1_b, fc2_w, fc2_b, fc3_w, fc3_b)
  return out8[:, 0]


# speedup vs baseline: 1.6030x; 1.2727x over previous
import functools

import jax
import jax.numpy as jnp
from jax import lax
from jax.experimental import pallas as pl
from jax.experimental.pallas import tpu as pltpu
from jax.experimental.pallas import tpu_sc as plsc

NUM_USERS = 25000
NUM_ITEMS = 25000
N = NUM_USERS + NUM_ITEMS
D = 64
DH = D // 2
NNZ = 800000
B = 4096

NC = 2
NS = 16
N_P = 50176
R_T = N_P // NS

G = 128
GPS = 28
SUP = GPS * G
NSUP = 14
EPS = NSUP * SUP
NNZ_P = NS * EPS


def _make_spmm():
  mesh = plsc.VectorSubcoreMesh(core_axis_name="c", subcore_axis_name="s")

  @functools.partial(
      pl.kernel,
      out_type=jax.ShapeDtypeStruct((NC * N_P, DH), jnp.float32),
      mesh=mesh,
      compiler_params=pltpu.CompilerParams(use_tc_tiling_on_sc=False),
      scratch_types=[
          pltpu.VMEM((SUP,), jnp.int32),
          pltpu.VMEM((SUP,), jnp.int32),
          pltpu.VMEM((SUP,), jnp.float32),
          pltpu.VMEM((4, G), jnp.int32),
          pltpu.VMEM((G, DH), jnp.float32),
          pltpu.VMEM((G, DH), jnp.float32),
          pltpu.VMEM((G, DH), jnp.float32),
          pltpu.VMEM((G, DH), jnp.float32),
          pltpu.VMEM_SHARED((N_P, DH), jnp.float32),
          pltpu.SemaphoreType.DMA,
          pltpu.SemaphoreType.DMA,
          pltpu.SemaphoreType.DMA,
          pltpu.SemaphoreType.DMA,
          pltpu.SemaphoreType.DMA,
          pltpu.SemaphoreType.DMA,
          pltpu.SemaphoreType.DMA,
          pltpu.SemaphoreType.DMA,
      ],
  )
  def spmm(feats_hbm, rows_hbm, cols_hbm, vals_hbm, z_hbm, out_hbm,
           rows_v, cols_v, vals_v, lidx_v, gbuf_0, gbuf_1, gbuf_2, gbuf_3,
           acc, gsem_0, gsem_1, gsem_2, gsem_3,
           ssem_0, ssem_1, ssem_2, ssem_3):
    c = lax.axis_index("c")
    s = lax.axis_index("s")
    tbase = c * N_P

    pltpu.sync_copy(z_hbm, acc.at[pl.ds(s * R_T, R_T)])
    plsc.subcore_barrier()

    bufs = ((gbuf_0, gsem_0, ssem_0), (gbuf_1, gsem_1, ssem_1),
            (gbuf_2, gsem_2, ssem_2), (gbuf_3, gsem_3, ssem_3))

    def issue(g, buf, sem):
      pltpu.async_copy(feats_hbm.at[cols_v.at[pl.ds(g * G, G)]], buf, sem)

    def drain(buf, sem):
      pltpu.make_async_copy(feats_hbm.at[pl.ds(0, G)], buf, sem).wait()

    def process(g, buf, bsel):
      @plsc.parallel_loop(0, G // 16)
      def _(i):
        lidx_v[bsel, pl.ds(i * 16, 16)] = rows_v[pl.ds(g * G + i * 16, 16)]
      @plsc.parallel_loop(0, G // 16, unroll=2)
      def _(i):
        v16 = vals_v[pl.ds(g * G + i * 16, 16)]
        for k in range(16):
          e = i * 16 + k
          v = v16[k]
          for j in range(DH // 16):
            buf[e, pl.ds(j * 16, 16)] = buf[e, pl.ds(j * 16, 16)] * v
      pltpu.async_copy(buf, acc.at[lidx_v.at[bsel]], bufs[bsel][2], add=True)

    ebase = s * EPS

    def super_body(sup, _):
      eoff = ebase + sup * SUP
      pltpu.sync_copy(rows_hbm.at[pl.ds(eoff, SUP)], rows_v)
      pltpu.sync_copy(cols_hbm.at[pl.ds(eoff, SUP)], cols_v)
      pltpu.sync_copy(vals_hbm.at[pl.ds(eoff, SUP)], vals_v)

      @plsc.parallel_loop(0, SUP // 16, unroll=4)
      def _(i):
        cols_v[pl.ds(i * 16, 16)] = cols_v[pl.ds(i * 16, 16)] + tbase

      issue(0, gbuf_0, gsem_0)
      issue(1, gbuf_1, gsem_1)

      def quad(i, _):
        for b in range(4):
          g = i * 4 + b
          buf, sem, _ssem = bufs[b]
          nbuf, ngsem, nssem = bufs[(b + 2) % 4]
          drain(buf, sem)

          @pl.when((g >= 2) & (g + 2 < GPS))
          def _():
            drain(nbuf, nssem)

          @pl.when(g + 2 < GPS)
          def _():
            issue(g + 2, nbuf, ngsem)

          process(g, buf, b)
        return 0

      lax.fori_loop(0, GPS // 4, quad, 0)
      for b in range(4):
        drain(bufs[b][0], bufs[b][2])
      return 0

    lax.fori_loop(0, NSUP, super_body, 0)

    plsc.subcore_barrier()
    pltpu.sync_copy(acc.at[pl.ds(s * R_T, R_T)],
                    out_hbm.at[pl.ds(c * N_P + s * R_T, R_T)])

  return spmm


def _dense_layer(le_flat, f, w1, b1, w2, b2):
  blk = 512

  def body(lo_ref, hi_ref, f_ref, w1_ref, b1_ref, w2_ref, b2_ref,
           o_ref, os_ref):
    le_v = jnp.concatenate([lo_ref[...], hi_ref[...]], axis=-1)
    f_v = f_ref[...]
    acc = jnp.dot(le_v + f_v, w1_ref[...], preferred_element_type=jnp.float32)
    acc += jnp.dot(le_v * f_v, w2_ref[...], preferred_element_type=jnp.float32)
    acc += b1_ref[...] + b2_ref[...]
    x = jnp.where(acc > 0, acc, 0.01 * acc)
    o_ref[...] = x
    os_ref[0] = x[:, :DH]
    os_ref[1] = x[:, DH:]

  return pl.pallas_call(
      body,
      grid=(N_P // blk,),
      in_specs=[
          pl.BlockSpec((blk, DH), lambda i: (i, 0)),
          pl.BlockSpec((blk, DH), lambda i: (N_P // 512 + i, 0)),
          pl.BlockSpec((blk, D), lambda i: (i, 0)),
          pl.BlockSpec((D, D), lambda i: (0, 0)),
          pl.BlockSpec((1, D), lambda i: (0, 0)),
          pl.BlockSpec((D, D), lambda i: (0, 0)),
          pl.BlockSpec((1, D), lambda i: (0, 0)),
      ],
      out_specs=[
          pl.BlockSpec((blk, D), lambda i: (i, 0)),
          pl.BlockSpec((2, blk, DH), lambda i: (0, i, 0)),
      ],
      out_shape=[
          jax.ShapeDtypeStruct((N_P, D), jnp.float32),
          jax.ShapeDtypeStruct((2, N_P, DH), jnp.float32),
      ],
  )(le_flat, le_flat, f, w1, b1.reshape(1, D), w2, b2.reshape(1, D))


def _make_gather():
  mesh = plsc.VectorSubcoreMesh(core_axis_name="c", subcore_axis_name="s")
  ch = B // (NC * NS)

  @functools.partial(
      pl.kernel,
      out_type=[jax.ShapeDtypeStruct((B, D), jnp.float32)] * 6,
      mesh=mesh,
      compiler_params=pltpu.CompilerParams(use_tc_tiling_on_sc=False),
      scratch_types=[
          pltpu.VMEM((ch,), jnp.int32),
          pltpu.VMEM((ch, D), jnp.float32),
          pltpu.SemaphoreType.DMA,
      ],
  )
  def gather(f0, f1, f2, u_hbm, i_hbm, o0, o1, o2, o3, o4, o5,
             idx_v, buf, sem):
    c = lax.axis_index("c")
    s = lax.axis_index("s")
    base = (s * NC + c) * ch

    pltpu.sync_copy(u_hbm.at[pl.ds(base, ch)], idx_v)
    for f, o in ((f0, o0), (f1, o1), (f2, o2)):
      pltpu.async_copy(f.at[idx_v], buf, sem).wait()
      pltpu.sync_copy(buf, o.at[pl.ds(base, ch)])

    pltpu.sync_copy(i_hbm.at[pl.ds(base, ch)], idx_v)
    for i in range(ch // 16):
      idx_v[pl.ds(i * 16, 16)] = idx_v[pl.ds(i * 16, 16)] + NUM_USERS
    for f, o in ((f0, o3), (f1, o4), (f2, o5)):
      pltpu.async_copy(f.at[idx_v], buf, sem).wait()
      pltpu.sync_copy(buf, o.at[pl.ds(base, ch)])

  return gather


def _mlp(parts, fc1_w, fc1_b, fc2_w, fc2_b, fc3_w, fc3_b):
  blk = 512
  fc3_wp = jnp.concatenate([fc3_w, jnp.zeros((32, 7), jnp.float32)], axis=1)
  fc3_bp = jnp.concatenate([fc3_b, jnp.zeros((7,), jnp.float32)])

  def body(p0, p1, p2, p3, p4, p5, w1_ref, b1_ref, w2_ref, b2_ref,
           w3_ref, b3_ref, o_ref):
    w1 = w1_ref[...]
    acc = jnp.zeros((blk, 64), jnp.float32)
    for i, p in enumerate((p0, p1, p2, p3, p4, p5)):
      acc += jnp.dot(p[...], w1[i * 64:(i + 1) * 64, :],
                     preferred_element_type=jnp.float32)
    h1 = jnp.maximum(acc + b1_ref[...], 0.0)
    h2 = jnp.maximum(
        jnp.dot(h1, w2_ref[...], preferred_element_type=jnp.float32)
        + b2_ref[...], 0.0)
    o_ref[...] = (jnp.dot(h2, w3_ref[...], preferred_element_type=jnp.float32)
                  + b3_ref[...])

  part_spec = pl.BlockSpec((blk, D), lambda i: (i, 0))
  return pl.pallas_call(
      body,
      grid=(B // blk,),
      in_specs=[part_spec] * 6 + [
          pl.BlockSpec((6 * D, D), lambda i: (0, 0)),
          pl.BlockSpec((1, D), lambda i: (0, 0)),
          pl.BlockSpec((D, 32), lambda i: (0, 0)),
          pl.BlockSpec((1, 32), lambda i: (0, 0)),
          pl.BlockSpec((32, 8), lambda i: (0, 0)),
          pl.BlockSpec((1, 8), lambda i: (0, 0)),
      ],
      out_specs=pl.BlockSpec((blk, 8), lambda i: (i, 0)),
      out_shape=jax.ShapeDtypeStruct((B, 8), jnp.float32),
  )(*parts, fc1_w, fc1_b.reshape(1, D), fc2_w, fc2_b.reshape(1, 32),
    fc3_wp, fc3_bp.reshape(1, 8))


def kernel(uids, iids, user_emb, item_emb, L_rows, L_cols, L_vals,
           W1_0, b1_0, W2_0, b2_0, W1_1, b1_1, W2_1, b2_1,
           fc1_w, fc1_b, fc2_w, fc2_b, fc3_w, fc3_b):
  feats0 = jnp.concatenate(
      [user_emb, item_emb, jnp.zeros((N_P - N, D), jnp.float32)], axis=0)
  feats0_flat = jnp.concatenate([feats0[:, :DH], feats0[:, DH:]], axis=0)
  pad_e = NNZ_P - NNZ
  rows_p = jnp.concatenate(
      [L_rows.astype(jnp.int32), jnp.zeros((pad_e,), jnp.int32)])
  cols_p = jnp.concatenate(
      [L_cols.astype(jnp.int32), jnp.zeros((pad_e,), jnp.int32)])
  vals_p = jnp.concatenate(
      [L_vals, jnp.zeros((pad_e,), jnp.float32)])
  ztile = jnp.zeros((R_T, DH), jnp.float32)

  spmm = _make_spmm()
  le0 = spmm(feats0_flat, rows_p, cols_p, vals_p, ztile)
  f1, f1_split = _dense_layer(le0, feats0, W1_0, b1_0, W2_0, b2_0)
  le1 = spmm(f1_split.reshape(NC * N_P, DH), rows_p, cols_p, vals_p, ztile)
  f2, _ = _dense_layer(le1, f1, W1_1, b1_1, W2_1, b2_1)

  parts = _make_gather()(feats0, f1, f2,
                         uids.astype(jnp.int32), iids.astype(jnp.int32))
  out8 = _mlp(parts, fc1_w, fc1_b, fc2_w, fc2_b, fc3_w, fc3_b)
  return out8[:, 0]


# speedup vs baseline: 1.6060x; 1.0018x over previous
import functools

import jax
import jax.numpy as jnp
from jax import lax
from jax.experimental import pallas as pl
from jax.experimental.pallas import tpu as pltpu
from jax.experimental.pallas import tpu_sc as plsc

NUM_USERS = 25000
NUM_ITEMS = 25000
N = NUM_USERS + NUM_ITEMS
D = 64
DH = D // 2
NNZ = 800000
B = 4096

NC = 2
NS = 16
N_P = 50176
R_T = N_P // NS

G = 128
GPS = 28
SUP = GPS * G
NSUP = 14
EPS = NSUP * SUP
NNZ_P = NS * EPS


def _make_spmm():
  mesh = plsc.VectorSubcoreMesh(core_axis_name="c", subcore_axis_name="s")

  @functools.partial(
      pl.kernel,
      out_type=jax.ShapeDtypeStruct((NC * N_P, DH), jnp.float32),
      mesh=mesh,
      compiler_params=pltpu.CompilerParams(use_tc_tiling_on_sc=False),
      scratch_types=[
          pltpu.VMEM((SUP,), jnp.int32),
          pltpu.VMEM((SUP,), jnp.int32),
          pltpu.VMEM((SUP,), jnp.float32),
          pltpu.VMEM((4, G), jnp.int32),
          pltpu.VMEM((G, DH), jnp.float32),
          pltpu.VMEM((G, DH), jnp.float32),
          pltpu.VMEM((G, DH), jnp.float32),
          pltpu.VMEM((G, DH), jnp.float32),
          pltpu.VMEM_SHARED((N_P, DH), jnp.float32),
          pltpu.SemaphoreType.DMA,
          pltpu.SemaphoreType.DMA,
          pltpu.SemaphoreType.DMA,
          pltpu.SemaphoreType.DMA,
          pltpu.SemaphoreType.DMA,
          pltpu.SemaphoreType.DMA,
          pltpu.SemaphoreType.DMA,
          pltpu.SemaphoreType.DMA,
      ],
  )
  def spmm(feats_hbm, rows_hbm, cols_hbm, vals_hbm, z_hbm, out_hbm,
           rows_v, cols_v, vals_v, lidx_v, gbuf_0, gbuf_1, gbuf_2, gbuf_3,
           acc, gsem_0, gsem_1, gsem_2, gsem_3,
           ssem_0, ssem_1, ssem_2, ssem_3):
    c = lax.axis_index("c")
    s = lax.axis_index("s")
    tbase = c * N_P

    pltpu.sync_copy(z_hbm, acc.at[pl.ds(s * R_T, R_T)])
    plsc.subcore_barrier()

    bufs = ((gbuf_0, gsem_0, ssem_0), (gbuf_1, gsem_1, ssem_1),
            (gbuf_2, gsem_2, ssem_2), (gbuf_3, gsem_3, ssem_3))

    def issue(g, buf, sem):
      pltpu.async_copy(feats_hbm.at[cols_v.at[pl.ds(g * G, G)]], buf, sem)

    def drain(buf, sem):
      pltpu.make_async_copy(feats_hbm.at[pl.ds(0, G)], buf, sem).wait()

    def process(g, buf, bsel):
      @plsc.parallel_loop(0, G // 16)
      def _(i):
        lidx_v[bsel, pl.ds(i * 16, 16)] = rows_v[pl.ds(g * G + i * 16, 16)]
      @plsc.parallel_loop(0, G // 16, unroll=2)
      def _(i):
        v16 = vals_v[pl.ds(g * G + i * 16, 16)]
        for k in range(16):
          e = i * 16 + k
          v = v16[k]
          for j in range(DH // 16):
            buf[e, pl.ds(j * 16, 16)] = buf[e, pl.ds(j * 16, 16)] * v
      pltpu.async_copy(buf, acc.at[lidx_v.at[bsel]], bufs[bsel][2], add=True)

    ebase = s * EPS

    def super_body(sup, _):
      eoff = ebase + sup * SUP
      pltpu.sync_copy(rows_hbm.at[pl.ds(eoff, SUP)], rows_v)
      pltpu.sync_copy(cols_hbm.at[pl.ds(eoff, SUP)], cols_v)
      pltpu.sync_copy(vals_hbm.at[pl.ds(eoff, SUP)], vals_v)

      @plsc.parallel_loop(0, SUP // 16, unroll=4)
      def _(i):
        cols_v[pl.ds(i * 16, 16)] = cols_v[pl.ds(i * 16, 16)] + tbase

      issue(0, gbuf_0, gsem_0)
      issue(1, gbuf_1, gsem_1)
      issue(2, gbuf_2, gsem_2)

      def quad(i, _):
        for b in range(4):
          g = i * 4 + b
          buf, sem, _ssem = bufs[b]
          nbuf, ngsem, nssem = bufs[(b + 3) % 4]
          drain(buf, sem)

          @pl.when((g >= 1) & (g + 3 < GPS))
          def _():
            drain(nbuf, nssem)

          @pl.when(g + 3 < GPS)
          def _():
            issue(g + 3, nbuf, ngsem)

          process(g, buf, b)
        return 0

      lax.fori_loop(0, GPS // 4, quad, 0)
      for b in range(4):
        drain(bufs[b][0], bufs[b][2])
      return 0

    lax.fori_loop(0, NSUP, super_body, 0)

    plsc.subcore_barrier()
    pltpu.sync_copy(acc.at[pl.ds(s * R_T, R_T)],
                    out_hbm.at[pl.ds(c * N_P + s * R_T, R_T)])

  return spmm


def _dense_layer(le_flat, f, w1, b1, w2, b2):
  blk = 512

  def body(lo_ref, hi_ref, f_ref, w1_ref, b1_ref, w2_ref, b2_ref,
           o_ref, os_ref):
    le_v = jnp.concatenate([lo_ref[...], hi_ref[...]], axis=-1)
    f_v = f_ref[...]
    acc = jnp.dot(le_v + f_v, w1_ref[...], preferred_element_type=jnp.float32)
    acc += jnp.dot(le_v * f_v, w2_ref[...], preferred_element_type=jnp.float32)
    acc += b1_ref[...] + b2_ref[...]
    x = jnp.where(acc > 0, acc, 0.01 * acc)
    o_ref[...] = x
    os_ref[0] = x[:, :DH]
    os_ref[1] = x[:, DH:]

  return pl.pallas_call(
      body,
      grid=(N_P // blk,),
      in_specs=[
          pl.BlockSpec((blk, DH), lambda i: (i, 0)),
          pl.BlockSpec((blk, DH), lambda i: (N_P // 512 + i, 0)),
          pl.BlockSpec((blk, D), lambda i: (i, 0)),
          pl.BlockSpec((D, D), lambda i: (0, 0)),
          pl.BlockSpec((1, D), lambda i: (0, 0)),
          pl.BlockSpec((D, D), lambda i: (0, 0)),
          pl.BlockSpec((1, D), lambda i: (0, 0)),
      ],
      out_specs=[
          pl.BlockSpec((blk, D), lambda i: (i, 0)),
          pl.BlockSpec((2, blk, DH), lambda i: (0, i, 0)),
      ],
      out_shape=[
          jax.ShapeDtypeStruct((N_P, D), jnp.float32),
          jax.ShapeDtypeStruct((2, N_P, DH), jnp.float32),
      ],
  )(le_flat, le_flat, f, w1, b1.reshape(1, D), w2, b2.reshape(1, D))


def _make_gather():
  mesh = plsc.VectorSubcoreMesh(core_axis_name="c", subcore_axis_name="s")
  ch = B // (NC * NS)

  @functools.partial(
      pl.kernel,
      out_type=[jax.ShapeDtypeStruct((B, D), jnp.float32)] * 6,
      mesh=mesh,
      compiler_params=pltpu.CompilerParams(use_tc_tiling_on_sc=False),
      scratch_types=[
          pltpu.VMEM((ch,), jnp.int32),
          pltpu.VMEM((ch, D), jnp.float32),
          pltpu.SemaphoreType.DMA,
      ],
  )
  def gather(f0, f1, f2, u_hbm, i_hbm, o0, o1, o2, o3, o4, o5,
             idx_v, buf, sem):
    c = lax.axis_index("c")
    s = lax.axis_index("s")
    base = (s * NC + c) * ch

    pltpu.sync_copy(u_hbm.at[pl.ds(base, ch)], idx_v)
    for f, o in ((f0, o0), (f1, o1), (f2, o2)):
      pltpu.async_copy(f.at[idx_v], buf, sem).wait()
      pltpu.sync_copy(buf, o.at[pl.ds(base, ch)])

    pltpu.sync_copy(i_hbm.at[pl.ds(base, ch)], idx_v)
    for i in range(ch // 16):
      idx_v[pl.ds(i * 16, 16)] = idx_v[pl.ds(i * 16, 16)] + NUM_USERS
    for f, o in ((f0, o3), (f1, o4), (f2, o5)):
      pltpu.async_copy(f.at[idx_v], buf, sem).wait()
      pltpu.sync_copy(buf, o.at[pl.ds(base, ch)])

  return gather


def _mlp(parts, fc1_w, fc1_b, fc2_w, fc2_b, fc3_w, fc3_b):
  blk = 512
  fc3_wp = jnp.concatenate([fc3_w, jnp.zeros((32, 7), jnp.float32)], axis=1)
  fc3_bp = jnp.concatenate([fc3_b, jnp.zeros((7,), jnp.float32)])

  def body(p0, p1, p2, p3, p4, p5, w1_ref, b1_ref, w2_ref, b2_ref,
           w3_ref, b3_ref, o_ref):
    w1 = w1_ref[...]
    acc = jnp.zeros((blk, 64), jnp.float32)
    for i, p in enumerate((p0, p1, p2, p3, p4, p5)):
      acc += jnp.dot(p[...], w1[i * 64:(i + 1) * 64, :],
                     preferred_element_type=jnp.float32)
    h1 = jnp.maximum(acc + b1_ref[...], 0.0)
    h2 = jnp.maximum(
        jnp.dot(h1, w2_ref[...], preferred_element_type=jnp.float32)
        + b2_ref[...], 0.0)
    o_ref[...] = (jnp.dot(h2, w3_ref[...], preferred_element_type=jnp.float32)
                  + b3_ref[...])

  part_spec = pl.BlockSpec((blk, D), lambda i: (i, 0))
  return pl.pallas_call(
      body,
      grid=(B // blk,),
      in_specs=[part_spec] * 6 + [
          pl.BlockSpec((6 * D, D), lambda i: (0, 0)),
          pl.BlockSpec((1, D), lambda i: (0, 0)),
          pl.BlockSpec((D, 32), lambda i: (0, 0)),
          pl.BlockSpec((1, 32), lambda i: (0, 0)),
          pl.BlockSpec((32, 8), lambda i: (0, 0)),
          pl.BlockSpec((1, 8), lambda i: (0, 0)),
      ],
      out_specs=pl.BlockSpec((blk, 8), lambda i: (i, 0)),
      out_shape=jax.ShapeDtypeStruct((B, 8), jnp.float32),
  )(*parts, fc1_w, fc1_b.reshape(1, D), fc2_w, fc2_b.reshape(1, 32),
    fc3_wp, fc3_bp.reshape(1, 8))


def kernel(uids, iids, user_emb, item_emb, L_rows, L_cols, L_vals,
           W1_0, b1_0, W2_0, b2_0, W1_1, b1_1, W2_1, b2_1,
           fc1_w, fc1_b, fc2_w, fc2_b, fc3_w, fc3_b):
  feats0 = jnp.concatenate(
      [user_emb, item_emb, jnp.zeros((N_P - N, D), jnp.float32)], axis=0)
  feats0_flat = jnp.concatenate([feats0[:, :DH], feats0[:, DH:]], axis=0)
  pad_e = NNZ_P - NNZ
  rows_p = jnp.concatenate(
      [L_rows.astype(jnp.int32), jnp.zeros((pad_e,), jnp.int32)])
  cols_p = jnp.concatenate(
      [L_cols.astype(jnp.int32), jnp.zeros((pad_e,), jnp.int32)])
  vals_p = jnp.concatenate(
      [L_vals, jnp.zeros((pad_e,), jnp.float32)])
  ztile = jnp.zeros((R_T, DH), jnp.float32)

  spmm = _make_spmm()
  le0 = spmm(feats0_flat, rows_p, cols_p, vals_p, ztile)
  f1, f1_split = _dense_layer(le0, feats0, W1_0, b1_0, W2_0, b2_0)
  le1 = spmm(f1_split.reshape(NC * N_P, DH), rows_p, cols_p, vals_p, ztile)
  f2, _ = _dense_layer(le1, f1, W1_1, b1_1, W2_1, b2_1)

  parts = _make_gather()(feats0, f1, f2,
                         uids.astype(jnp.int32), iids.astype(jnp.int32))
  out8 = _mlp(parts, fc1_w, fc1_b, fc2_w, fc2_b, fc3_w, fc3_b)
  return out8[:, 0]


# speedup vs baseline: 1.7829x; 1.1101x over previous
import functools

import jax
import jax.numpy as jnp
from jax import lax
from jax.experimental import pallas as pl
from jax.experimental.pallas import tpu as pltpu
from jax.experimental.pallas import tpu_sc as plsc

NUM_USERS = 25000
NUM_ITEMS = 25000
N = NUM_USERS + NUM_ITEMS
D = 64
DH = D // 2
NNZ = 800000
B = 4096

NC = 2
NS = 16
N_P = 50176
R_T = N_P // NS

G = 128
GPS = 28
SUP = GPS * G
NSUP = 14
EPS = NSUP * SUP
NNZ_P = NS * EPS


def _make_spmm():
  mesh = plsc.VectorSubcoreMesh(core_axis_name="c", subcore_axis_name="s")

  @functools.partial(
      pl.kernel,
      out_type=jax.ShapeDtypeStruct((NC * N_P, DH), jnp.float32),
      mesh=mesh,
      compiler_params=pltpu.CompilerParams(use_tc_tiling_on_sc=False),
      scratch_types=[
          pltpu.VMEM((SUP,), jnp.int32),
          pltpu.VMEM((SUP,), jnp.int32),
          pltpu.VMEM((SUP,), jnp.float32),
          pltpu.VMEM((4, G), jnp.int32),
          pltpu.VMEM((G, DH), jnp.float32),
          pltpu.VMEM((G, DH), jnp.float32),
          pltpu.VMEM((G, DH), jnp.float32),
          pltpu.VMEM((G, DH), jnp.float32),
          pltpu.VMEM_SHARED((N_P, DH), jnp.float32),
          pltpu.SemaphoreType.DMA,
          pltpu.SemaphoreType.DMA,
          pltpu.SemaphoreType.DMA,
          pltpu.SemaphoreType.DMA,
          pltpu.SemaphoreType.DMA,
          pltpu.SemaphoreType.DMA,
          pltpu.SemaphoreType.DMA,
          pltpu.SemaphoreType.DMA,
      ],
  )
  def spmm(feats_hbm, rows_hbm, cols_hbm, vals_hbm, z_hbm, out_hbm,
           rows_v, cols_v, vals_v, lidx_v, gbuf_0, gbuf_1, gbuf_2, gbuf_3,
           acc, gsem_0, gsem_1, gsem_2, gsem_3,
           ssem_0, ssem_1, ssem_2, ssem_3):
    c = lax.axis_index("c")
    s = lax.axis_index("s")
    tbase = c * N_P

    pltpu.sync_copy(z_hbm, acc.at[pl.ds(s * R_T, R_T)])
    plsc.subcore_barrier()

    bufs = ((gbuf_0, gsem_0, ssem_0), (gbuf_1, gsem_1, ssem_1),
            (gbuf_2, gsem_2, ssem_2), (gbuf_3, gsem_3, ssem_3))

    def issue(g, buf, sem):
      pltpu.async_copy(feats_hbm.at[cols_v.at[pl.ds(g * G, G)]], buf, sem)

    def drain(buf, sem):
      pltpu.make_async_copy(feats_hbm.at[pl.ds(0, G)], buf, sem).wait()

    def process(g, buf, bsel):
      @plsc.parallel_loop(0, G // 16)
      def _(i):
        lidx_v[bsel, pl.ds(i * 16, 16)] = rows_v[pl.ds(g * G + i * 16, 16)]
      @plsc.parallel_loop(0, G // 16, unroll=2)
      def _(i):
        v16 = vals_v[pl.ds(g * G + i * 16, 16)]
        for k in range(16):
          e = i * 16 + k
          v = v16[k]
          for j in range(DH // 16):
            buf[e, pl.ds(j * 16, 16)] = buf[e, pl.ds(j * 16, 16)] * v
      pltpu.async_copy(buf, acc.at[lidx_v.at[bsel]], bufs[bsel][2], add=True)

    ebase = s * EPS

    def super_body(sup, _):
      eoff = ebase + sup * SUP
      pltpu.sync_copy(rows_hbm.at[pl.ds(eoff, SUP)], rows_v)
      pltpu.sync_copy(cols_hbm.at[pl.ds(eoff, SUP)], cols_v)
      pltpu.sync_copy(vals_hbm.at[pl.ds(eoff, SUP)], vals_v)

      @plsc.parallel_loop(0, SUP // 16, unroll=4)
      def _(i):
        cols_v[pl.ds(i * 16, 16)] = cols_v[pl.ds(i * 16, 16)] + tbase

      issue(0, gbuf_0, gsem_0)
      issue(1, gbuf_1, gsem_1)
      issue(2, gbuf_2, gsem_2)

      def quad(i, _):
        for b in range(4):
          g = i * 4 + b
          buf, sem, _ssem = bufs[b]
          nbuf, ngsem, nssem = bufs[(b + 3) % 4]
          drain(buf, sem)

          @pl.when((g >= 1) & (g + 3 < GPS))
          def _():
            drain(nbuf, nssem)

          @pl.when(g + 3 < GPS)
          def _():
            issue(g + 3, nbuf, ngsem)

          process(g, buf, b)
        return 0

      lax.fori_loop(0, GPS // 4, quad, 0)
      for b in range(4):
        drain(bufs[b][0], bufs[b][2])
      return 0

    lax.fori_loop(0, NSUP, super_body, 0)

    plsc.subcore_barrier()
    pltpu.sync_copy(acc.at[pl.ds(s * R_T, R_T)],
                    out_hbm.at[pl.ds(c * N_P + s * R_T, R_T)])

  return spmm


def _dense_layer(le_flat, f, w1, b1, w2, b2):
  blk = 3136

  def body(lo_ref, hi_ref, f_ref, w1_ref, b1_ref, w2_ref, b2_ref,
           o_ref, os_ref):
    le_v = jnp.concatenate([lo_ref[...], hi_ref[...]], axis=-1)
    f_v = f_ref[...]
    acc = jnp.dot(le_v + f_v, w1_ref[...], preferred_element_type=jnp.float32)
    acc += jnp.dot(le_v * f_v, w2_ref[...], preferred_element_type=jnp.float32)
    acc += b1_ref[...] + b2_ref[...]
    x = jnp.where(acc > 0, acc, 0.01 * acc)
    o_ref[...] = x
    os_ref[0] = x[:, :DH]
    os_ref[1] = x[:, DH:]

  return pl.pallas_call(
      body,
      grid=(N_P // blk,),
      in_specs=[
          pl.BlockSpec((blk, DH), lambda i: (i, 0)),
          pl.BlockSpec((blk, DH), lambda i: (N_P // 3136 + i, 0)),
          pl.BlockSpec((blk, D), lambda i: (i, 0)),
          pl.BlockSpec((D, D), lambda i: (0, 0)),
          pl.BlockSpec((1, D), lambda i: (0, 0)),
          pl.BlockSpec((D, D), lambda i: (0, 0)),
          pl.BlockSpec((1, D), lambda i: (0, 0)),
      ],
      out_specs=[
          pl.BlockSpec((blk, D), lambda i: (i, 0)),
          pl.BlockSpec((2, blk, DH), lambda i: (0, i, 0)),
      ],
      out_shape=[
          jax.ShapeDtypeStruct((N_P, D), jnp.float32),
          jax.ShapeDtypeStruct((2, N_P, DH), jnp.float32),
      ],
  )(le_flat, le_flat, f, w1, b1.reshape(1, D), w2, b2.reshape(1, D))


def _make_gather():
  mesh = plsc.VectorSubcoreMesh(core_axis_name="c", subcore_axis_name="s")
  ch = B // (NC * NS)

  @functools.partial(
      pl.kernel,
      out_type=[jax.ShapeDtypeStruct((B, D), jnp.float32)] * 6,
      mesh=mesh,
      compiler_params=pltpu.CompilerParams(use_tc_tiling_on_sc=False),
      scratch_types=[
          pltpu.VMEM((ch,), jnp.int32),
          pltpu.VMEM((ch, D), jnp.float32),
          pltpu.SemaphoreType.DMA,
      ],
  )
  def gather(f0, f1, f2, u_hbm, i_hbm, o0, o1, o2, o3, o4, o5,
             idx_v, buf, sem):
    c = lax.axis_index("c")
    s = lax.axis_index("s")
    base = (s * NC + c) * ch

    pltpu.sync_copy(u_hbm.at[pl.ds(base, ch)], idx_v)
    for f, o in ((f0, o0), (f1, o1), (f2, o2)):
      pltpu.async_copy(f.at[idx_v], buf, sem).wait()
      pltpu.sync_copy(buf, o.at[pl.ds(base, ch)])

    pltpu.sync_copy(i_hbm.at[pl.ds(base, ch)], idx_v)
    for i in range(ch // 16):
      idx_v[pl.ds(i * 16, 16)] = idx_v[pl.ds(i * 16, 16)] + NUM_USERS
    for f, o in ((f0, o3), (f1, o4), (f2, o5)):
      pltpu.async_copy(f.at[idx_v], buf, sem).wait()
      pltpu.sync_copy(buf, o.at[pl.ds(base, ch)])

  return gather


def _mlp(parts, fc1_w, fc1_b, fc2_w, fc2_b, fc3_w, fc3_b):
  blk = 2048
  fc3_wp = jnp.concatenate([fc3_w, jnp.zeros((32, 7), jnp.float32)], axis=1)
  fc3_bp = jnp.concatenate([fc3_b, jnp.zeros((7,), jnp.float32)])

  def body(p0, p1, p2, p3, p4, p5, w1_ref, b1_ref, w2_ref, b2_ref,
           w3_ref, b3_ref, o_ref):
    w1 = w1_ref[...]
    acc = jnp.zeros((blk, 64), jnp.float32)
    for i, p in enumerate((p0, p1, p2, p3, p4, p5)):
      acc += jnp.dot(p[...], w1[i * 64:(i + 1) * 64, :],
                     preferred_element_type=jnp.float32)
    h1 = jnp.maximum(acc + b1_ref[...], 0.0)
    h2 = jnp.maximum(
        jnp.dot(h1, w2_ref[...], preferred_element_type=jnp.float32)
        + b2_ref[...], 0.0)
    o_ref[...] = (jnp.dot(h2, w3_ref[...], preferred_element_type=jnp.float32)
                  + b3_ref[...])

  part_spec = pl.BlockSpec((blk, D), lambda i: (i, 0))
  return pl.pallas_call(
      body,
      grid=(B // blk,),
      in_specs=[part_spec] * 6 + [
          pl.BlockSpec((6 * D, D), lambda i: (0, 0)),
          pl.BlockSpec((1, D), lambda i: (0, 0)),
          pl.BlockSpec((D, 32), lambda i: (0, 0)),
          pl.BlockSpec((1, 32), lambda i: (0, 0)),
          pl.BlockSpec((32, 8), lambda i: (0, 0)),
          pl.BlockSpec((1, 8), lambda i: (0, 0)),
      ],
      out_specs=pl.BlockSpec((blk, 8), lambda i: (i, 0)),
      out_shape=jax.ShapeDtypeStruct((B, 8), jnp.float32),
  )(*parts, fc1_w, fc1_b.reshape(1, D), fc2_w, fc2_b.reshape(1, 32),
    fc3_wp, fc3_bp.reshape(1, 8))


def kernel(uids, iids, user_emb, item_emb, L_rows, L_cols, L_vals,
           W1_0, b1_0, W2_0, b2_0, W1_1, b1_1, W2_1, b2_1,
           fc1_w, fc1_b, fc2_w, fc2_b, fc3_w, fc3_b):
  feats0 = jnp.concatenate(
      [user_emb, item_emb, jnp.zeros((N_P - N, D), jnp.float32)], axis=0)
  feats0_flat = jnp.concatenate([feats0[:, :DH], feats0[:, DH:]], axis=0)
  pad_e = NNZ_P - NNZ
  rows_p = jnp.concatenate(
      [L_rows.astype(jnp.int32), jnp.zeros((pad_e,), jnp.int32)])
  cols_p = jnp.concatenate(
      [L_cols.astype(jnp.int32), jnp.zeros((pad_e,), jnp.int32)])
  vals_p = jnp.concatenate(
      [L_vals, jnp.zeros((pad_e,), jnp.float32)])
  ztile = jnp.zeros((R_T, DH), jnp.float32)

  spmm = _make_spmm()
  le0 = spmm(feats0_flat, rows_p, cols_p, vals_p, ztile)
  f1, f1_split = _dense_layer(le0, feats0, W1_0, b1_0, W2_0, b2_0)
  le1 = spmm(f1_split.reshape(NC * N_P, DH), rows_p, cols_p, vals_p, ztile)
  f2, _ = _dense_layer(le1, f1, W1_1, b1_1, W2_1, b2_1)

  parts = _make_gather()(feats0, f1, f2,
                         uids.astype(jnp.int32), iids.astype(jnp.int32))
  out8 = _mlp(parts, fc1_w, fc1_b, fc2_w, fc2_b, fc3_w, fc3_b)
  return out8[:, 0]


# speedup vs baseline: 1.7927x; 1.0055x over previous
import functools

import jax
import jax.numpy as jnp
from jax import lax
from jax.experimental import pallas as pl
from jax.experimental.pallas import tpu as pltpu
from jax.experimental.pallas import tpu_sc as plsc

NUM_USERS = 25000
NUM_ITEMS = 25000
N = NUM_USERS + NUM_ITEMS
D = 64
DH = D // 2
NNZ = 800000
B = 4096

NC = 2
NS = 16
N_P = 50176
R_T = N_P // NS

G = 128
GPS = 28
SUP = GPS * G
NSUP = 14
EPS = NSUP * SUP
NNZ_P = NS * EPS


def _make_spmm():
  mesh = plsc.VectorSubcoreMesh(core_axis_name="c", subcore_axis_name="s")

  @functools.partial(
      pl.kernel,
      out_type=jax.ShapeDtypeStruct((NC, N_P, DH), jnp.float32),
      mesh=mesh,
      compiler_params=pltpu.CompilerParams(use_tc_tiling_on_sc=False),
      scratch_types=[
          pltpu.VMEM((SUP,), jnp.int32),
          pltpu.VMEM((SUP,), jnp.int32),
          pltpu.VMEM((SUP,), jnp.float32),
          pltpu.VMEM((4, G), jnp.int32),
          pltpu.VMEM((G, DH), jnp.float32),
          pltpu.VMEM((G, DH), jnp.float32),
          pltpu.VMEM((G, DH), jnp.float32),
          pltpu.VMEM((G, DH), jnp.float32),
          pltpu.VMEM_SHARED((N_P, DH), jnp.float32),
          pltpu.SemaphoreType.DMA,
          pltpu.SemaphoreType.DMA,
          pltpu.SemaphoreType.DMA,
          pltpu.SemaphoreType.DMA,
          pltpu.SemaphoreType.DMA,
          pltpu.SemaphoreType.DMA,
          pltpu.SemaphoreType.DMA,
          pltpu.SemaphoreType.DMA,
      ],
  )
  def spmm(feats_hbm, rows_hbm, cols_hbm, vals_hbm, z_hbm, out_hbm,
           rows_v, cols_v, vals_v, lidx_v, gbuf_0, gbuf_1, gbuf_2, gbuf_3,
           acc, gsem_0, gsem_1, gsem_2, gsem_3,
           ssem_0, ssem_1, ssem_2, ssem_3):
    c = lax.axis_index("c")
    s = lax.axis_index("s")

    pltpu.sync_copy(z_hbm, acc.at[pl.ds(s * R_T, R_T)])
    plsc.subcore_barrier()

    bufs = ((gbuf_0, gsem_0, ssem_0), (gbuf_1, gsem_1, ssem_1),
            (gbuf_2, gsem_2, ssem_2), (gbuf_3, gsem_3, ssem_3))

    def issue(g, buf, sem):
      pltpu.async_copy(feats_hbm.at[c].at[cols_v.at[pl.ds(g * G, G)]], buf, sem)

    def drain(buf, sem):
      pltpu.make_async_copy(feats_hbm.at[c].at[pl.ds(0, G)], buf, sem).wait()

    def process(g, buf, bsel):
      @plsc.parallel_loop(0, G // 16)
      def _(i):
        lidx_v[bsel, pl.ds(i * 16, 16)] = rows_v[pl.ds(g * G + i * 16, 16)]
      @plsc.parallel_loop(0, G // 16, unroll=2)
      def _(i):
        v16 = vals_v[pl.ds(g * G + i * 16, 16)]
        for k in range(16):
          e = i * 16 + k
          v = v16[k]
          for j in range(DH // 16):
            buf[e, pl.ds(j * 16, 16)] = buf[e, pl.ds(j * 16, 16)] * v
      pltpu.async_copy(buf, acc.at[lidx_v.at[bsel]], bufs[bsel][2], add=True)

    ebase = s * EPS

    def super_body(sup, _):
      eoff = ebase + sup * SUP
      pltpu.sync_copy(rows_hbm.at[pl.ds(eoff, SUP)], rows_v)
      pltpu.sync_copy(cols_hbm.at[pl.ds(eoff, SUP)], cols_v)
      pltpu.sync_copy(vals_hbm.at[pl.ds(eoff, SUP)], vals_v)

      issue(0, gbuf_0, gsem_0)
      issue(1, gbuf_1, gsem_1)
      issue(2, gbuf_2, gsem_2)

      def quad(i, _):
        for b in range(4):
          g = i * 4 + b
          buf, sem, _ssem = bufs[b]
          nbuf, ngsem, nssem = bufs[(b + 3) % 4]
          drain(buf, sem)

          @pl.when((g >= 1) & (g + 3 < GPS))
          def _():
            drain(nbuf, nssem)

          @pl.when(g + 3 < GPS)
          def _():
            issue(g + 3, nbuf, ngsem)

          process(g, buf, b)
        return 0

      lax.fori_loop(0, GPS // 4, quad, 0)
      for b in range(4):
        drain(bufs[b][0], bufs[b][2])
      return 0

    lax.fori_loop(0, NSUP, super_body, 0)

    plsc.subcore_barrier()
    pltpu.sync_copy(acc.at[pl.ds(s * R_T, R_T)],
                    out_hbm.at[c, pl.ds(s * R_T, R_T)])

  return spmm


def _dense_layer(le_3d, f, w1, b1, w2, b2):
  blk = 3136

  def body(le_ref, f_ref, w1_ref, b1_ref, w2_ref, b2_ref,
           o_ref, os_ref):
    le_v = jnp.concatenate([le_ref[0], le_ref[1]], axis=-1)
    f_v = f_ref[...]
    acc = jnp.dot(le_v + f_v, w1_ref[...], preferred_element_type=jnp.float32)
    acc += jnp.dot(le_v * f_v, w2_ref[...], preferred_element_type=jnp.float32)
    acc += b1_ref[...] + b2_ref[...]
    x = jnp.where(acc > 0, acc, 0.01 * acc)
    o_ref[...] = x
    os_ref[0] = x[:, :DH]
    os_ref[1] = x[:, DH:]

  return pl.pallas_call(
      body,
      grid=(N_P // blk,),
      in_specs=[
          pl.BlockSpec((2, blk, DH), lambda i: (0, i, 0)),
          pl.BlockSpec((blk, D), lambda i: (i, 0)),
          pl.BlockSpec((D, D), lambda i: (0, 0)),
          pl.BlockSpec((1, D), lambda i: (0, 0)),
          pl.BlockSpec((D, D), lambda i: (0, 0)),
          pl.BlockSpec((1, D), lambda i: (0, 0)),
      ],
      out_specs=[
          pl.BlockSpec((blk, D), lambda i: (i, 0)),
          pl.BlockSpec((2, blk, DH), lambda i: (0, i, 0)),
      ],
      out_shape=[
          jax.ShapeDtypeStruct((N_P, D), jnp.float32),
          jax.ShapeDtypeStruct((2, N_P, DH), jnp.float32),
      ],
  )(le_3d, f, w1, b1.reshape(1, D), w2, b2.reshape(1, D))


def _make_gather():
  mesh = plsc.VectorSubcoreMesh(core_axis_name="c", subcore_axis_name="s")
  ch = B // (NC * NS)

  @functools.partial(
      pl.kernel,
      out_type=[jax.ShapeDtypeStruct((B, D), jnp.float32)] * 6,
      mesh=mesh,
      compiler_params=pltpu.CompilerParams(use_tc_tiling_on_sc=False),
      scratch_types=[
          pltpu.VMEM((ch,), jnp.int32),
          pltpu.VMEM((ch, D), jnp.float32),
          pltpu.SemaphoreType.DMA,
      ],
  )
  def gather(f0, f1, f2, u_hbm, i_hbm, o0, o1, o2, o3, o4, o5,
             idx_v, buf, sem):
    c = lax.axis_index("c")
    s = lax.axis_index("s")
    base = (s * NC + c) * ch

    pltpu.sync_copy(u_hbm.at[pl.ds(base, ch)], idx_v)
    for f, o in ((f0, o0), (f1, o1), (f2, o2)):
      pltpu.async_copy(f.at[idx_v], buf, sem).wait()
      pltpu.sync_copy(buf, o.at[pl.ds(base, ch)])

    pltpu.sync_copy(i_hbm.at[pl.ds(base, ch)], idx_v)
    for i in range(ch // 16):
      idx_v[pl.ds(i * 16, 16)] = idx_v[pl.ds(i * 16, 16)] + NUM_USERS
    for f, o in ((f0, o3), (f1, o4), (f2, o5)):
      pltpu.async_copy(f.at[idx_v], buf, sem).wait()
      pltpu.sync_copy(buf, o.at[pl.ds(base, ch)])

  return gather


def _mlp(parts, fc1_w, fc1_b, fc2_w, fc2_b, fc3_w, fc3_b):
  blk = 2048
  fc3_wp = jnp.concatenate([fc3_w, jnp.zeros((32, 7), jnp.float32)], axis=1)
  fc3_bp = jnp.concatenate([fc3_b, jnp.zeros((7,), jnp.float32)])

  def body(p0, p1, p2, p3, p4, p5, w1_ref, b1_ref, w2_ref, b2_ref,
           w3_ref, b3_ref, o_ref):
    w1 = w1_ref[...]
    acc = jnp.zeros((blk, 64), jnp.float32)
    for i, p in enumerate((p0, p1, p2, p3, p4, p5)):
      acc += jnp.dot(p[...], w1[i * 64:(i + 1) * 64, :],
                     preferred_element_type=jnp.float32)
    h1 = jnp.maximum(acc + b1_ref[...], 0.0)
    h2 = jnp.maximum(
        jnp.dot(h1, w2_ref[...], preferred_element_type=jnp.float32)
        + b2_ref[...], 0.0)
    o_ref[...] = (jnp.dot(h2, w3_ref[...], preferred_element_type=jnp.float32)
                  + b3_ref[...])

  part_spec = pl.BlockSpec((blk, D), lambda i: (i, 0))
  return pl.pallas_call(
      body,
      grid=(B // blk,),
      in_specs=[part_spec] * 6 + [
          pl.BlockSpec((6 * D, D), lambda i: (0, 0)),
          pl.BlockSpec((1, D), lambda i: (0, 0)),
          pl.BlockSpec((D, 32), lambda i: (0, 0)),
          pl.BlockSpec((1, 32), lambda i: (0, 0)),
          pl.BlockSpec((32, 8), lambda i: (0, 0)),
          pl.BlockSpec((1, 8), lambda i: (0, 0)),
      ],
      out_specs=pl.BlockSpec((blk, 8), lambda i: (i, 0)),
      out_shape=jax.ShapeDtypeStruct((B, 8), jnp.float32),
  )(*parts, fc1_w, fc1_b.reshape(1, D), fc2_w, fc2_b.reshape(1, 32),
    fc3_wp, fc3_bp.reshape(1, 8))


def kernel(uids, iids, user_emb, item_emb, L_rows, L_cols, L_vals,
           W1_0, b1_0, W2_0, b2_0, W1_1, b1_1, W2_1, b2_1,
           fc1_w, fc1_b, fc2_w, fc2_b, fc3_w, fc3_b):
  feats0 = jnp.concatenate(
      [user_emb, item_emb, jnp.zeros((N_P - N, D), jnp.float32)], axis=0)
  feats0_split = jnp.stack([feats0[:, :DH], feats0[:, DH:]], axis=0)
  pad_e = NNZ_P - NNZ
  rows_p = jnp.concatenate(
      [L_rows.astype(jnp.int32), jnp.zeros((pad_e,), jnp.int32)])
  cols_p = jnp.concatenate(
      [L_cols.astype(jnp.int32), jnp.zeros((pad_e,), jnp.int32)])
  vals_p = jnp.concatenate(
      [L_vals, jnp.zeros((pad_e,), jnp.float32)])
  ztile = jnp.zeros((R_T, DH), jnp.float32)

  spmm = _make_spmm()
  le0 = spmm(feats0_split, rows_p, cols_p, vals_p, ztile)
  f1, f1_split = _dense_layer(le0, feats0, W1_0, b1_0, W2_0, b2_0)
  le1 = spmm(f1_split, rows_p, cols_p, vals_p, ztile)
  f2, _ = _dense_layer(le1, f1, W1_1, b1_1, W2_1, b2_1)

  parts = _make_gather()(feats0, f1, f2,
                         uids.astype(jnp.int32), iids.astype(jnp.int32))
  out8 = _mlp(parts, fc1_w, fc1_b, fc2_w, fc2_b, fc3_w, fc3_b)
  return out8[:, 0]
